# Initial kernel scaffold; baseline (speedup 1.0000x reference)
import jax
import jax.numpy as jnp
from jax.experimental import pallas as pl


def kernel(x, W1, b1, W2, b2, W3, b3, edge_src, edge_dst, edge_kidx):
    raise NotImplementedError("write your pallas kernel here")


# speedup vs baseline: 6.9472x; 6.9472x over previous
import functools

import jax
import jax.numpy as jnp
from jax import lax
from jax.experimental import pallas as pl
from jax.experimental.pallas import tpu as pltpu
from jax.experimental.pallas import tpu_sc as plsc

_N = 10000
_E = 160000
_NIN = 128
_CB = 32
_K = 27

_NC = 2
_NS = 16
_NW = _NC * _NS
_GB = 128
_EPT_G = -(-_E // (_NW * _GB))
_NG = _EPT_G
_EPAD = _NW * _NG * _GB
_RPT = 632
_NACC = _RPT * _NS
_NBUF = 6
_SLAG = 2

_BM = 1000


def _expand_body(x_ref, w1_ref, b1_ref, w2c_ref, g_ref):
    h = jnp.dot(x_ref[...], w1_ref[...], preferred_element_type=jnp.float32)
    h = jnp.maximum(h + b1_ref[...], 0.0)
    g_ref[...] = jnp.dot(h, w2c_ref[...], preferred_element_type=jnp.float32)


_expand = pl.pallas_call(
    _expand_body,
    grid=(_N // _BM,),
    in_specs=[
        pl.BlockSpec((_BM, _NIN), lambda i: (i, 0)),
        pl.BlockSpec((_NIN, _CB), lambda i: (0, 0)),
        pl.BlockSpec((1, _CB), lambda i: (0, 0)),
        pl.BlockSpec((_CB, _K * _CB), lambda i: (0, 0)),
    ],
    out_specs=pl.BlockSpec((_BM, _K * _CB), lambda i: (i, 0)),
    out_shape=jax.ShapeDtypeStruct((_N, _K * _CB), jnp.float32),
)


def _combine_body(p_ref, b2_ref, x_ref, w3_ref, b3_ref, o_ref):
    s = p_ref[0] + p_ref[1] + b2_ref[...]
    h2 = jnp.maximum(s, 0.0)
    h3 = jnp.dot(h2, w3_ref[...], preferred_element_type=jnp.float32)
    h3 = h3 + b3_ref[...]
    o_ref[...] = jnp.maximum((h3 + x_ref[...]) * 0.5, 0.0)


_combine = pl.pallas_call(
    _combine_body,
    grid=(_N // _BM,),
    in_specs=[
        pl.BlockSpec((_NC, _BM, _CB), lambda i: (0, i, 0)),
        pl.BlockSpec((1, _CB), lambda i: (0, 0)),
        pl.BlockSpec((_BM, _NIN), lambda i: (i, 0)),
        pl.BlockSpec((_CB, _NIN), lambda i: (0, 0)),
        pl.BlockSpec((1, _NIN), lambda i: (0, 0)),
    ],
    out_specs=pl.BlockSpec((_BM, _NIN), lambda i: (i, 0)),
    out_shape=jax.ShapeDtypeStruct((_N, _NIN), jnp.float32),
)


_mesh = plsc.VectorSubcoreMesh(core_axis_name="c", subcore_axis_name="s")


@functools.partial(
    pl.kernel,
    out_type=jax.ShapeDtypeStruct((_NC, _NACC, _CB), jnp.float32),
    mesh=_mesh,
    scratch_types=[
        pltpu.VMEM((_NG, _GB), jnp.int32),
        pltpu.VMEM((_NG, _GB), jnp.int32),
        pltpu.VMEM((_NBUF, _GB, _CB), jnp.float32),
        pltpu.VMEM((_RPT, _CB), jnp.float32),
        pltpu.VMEM_SHARED((_NACC, _CB), jnp.float32),
        pltpu.SemaphoreType.DMA,
        pltpu.SemaphoreType.DMA,
        pltpu.SemaphoreType.DMA,
    ],
    compiler_params=pltpu.CompilerParams(use_tc_tiling_on_sc=False),
)
def _sc_scatter(table, gidx, dst, out, gidx_v, dst_v, rows_v, stage_v,
                acc_sh, lsem, gsem, ssem):
    c = lax.axis_index("c")
    s = lax.axis_index("s")
    w = c * _NS + s

    def _z(i, carry):
        stage_v[i, pl.ds(0, 16)] = jnp.zeros((16,), jnp.float32)
        stage_v[i, pl.ds(16, 16)] = jnp.zeros((16,), jnp.float32)
        return carry
    lax.fori_loop(0, _RPT, _z, 0)
    pltpu.sync_copy(stage_v, acc_sh.at[pl.ds(s * _RPT, _RPT)])

    pltpu.async_copy(gidx.at[w], gidx_v, lsem)
    pltpu.async_copy(dst.at[w], dst_v, lsem).wait()
    pltpu.make_async_copy(gidx.at[w], gidx_v, lsem).wait()
    plsc.subcore_barrier()

    gds = [None] * _NG
    sds = [None] * _NG
    for b in range(min(_NBUF - _SLAG, _NG)):
        gds[b] = pltpu.async_copy(
            table.at[gidx_v.at[b]], rows_v.at[b % _NBUF], gsem)
    for g in range(_NG):
        buf = g % _NBUF
        gds[g].wait()
        sds[g] = pltpu.async_copy(
            rows_v.at[buf], acc_sh.at[dst_v.at[g]], ssem, add=True)
        nx = g + _NBUF - _SLAG
        if nx < _NG:
            prev = nx - _NBUF
            if prev >= 0:
                sds[prev].wait()
            gds[nx] = pltpu.async_copy(
                table.at[gidx_v.at[nx]], rows_v.at[nx % _NBUF], gsem)
    for g in range(max(0, _NG - _NBUF), _NG):
        sds[g].wait()

    plsc.subcore_barrier()

    pltpu.sync_copy(acc_sh.at[pl.ds(s * _RPT, _RPT)], stage_v)
    pltpu.sync_copy(stage_v, out.at[c, pl.ds(s * _RPT, _RPT)])


def kernel(x, W1, b1, W2, b2, W3, b3, edge_src, edge_dst, edge_kidx):
    w2cat = jnp.transpose(W2, (1, 0, 2)).reshape(_CB, _K * _CB)
    table = _expand(x, W1, b1.reshape(1, _CB), w2cat)

    pad = _EPAD - _E
    gidx = edge_src * _K + edge_kidx
    gidx = jnp.concatenate([gidx, jnp.zeros((pad,), jnp.int32)])
    dstp = jnp.concatenate([edge_dst, jnp.full((pad,), _N, jnp.int32)])
    gidx = gidx.reshape(_NW, _NG, _GB)
    dstp = dstp.reshape(_NW, _NG, _GB)

    parts = _sc_scatter(table.reshape(_N * _K, _CB), gidx, dstp)

    return _combine(parts[:, :_N, :], b2.reshape(1, _CB), x, W3,
                    b3.reshape(1, _NIN))
